# Initial kernel scaffold; baseline (speedup 1.0000x reference)
import jax
import jax.numpy as jnp
from jax.experimental import pallas as pl


def kernel(x, edge_index, W_gat, att_src, att_dst, b_gat, gamma, beta, W1, b1, W2, b2):
    raise NotImplementedError("write your pallas kernel here")


# speedup vs baseline: 63.6515x; 63.6515x over previous
import functools

import jax
import jax.numpy as jnp
from jax import lax
from jax.experimental import pallas as pl
from jax.experimental.pallas import tpu as pltpu
from jax.experimental.pallas import tpu_sc as plsc

DIM = 128
HEADS = 4
HEAD_DIM = DIM // HEADS
N_NODES = 10000
N_EDGES = 320000

NPAD = 10240
PADROW = N_NODES
ETOT = N_EDGES + N_NODES
NC, NS, NLANE = 2, 16, 16
NW = NC * NS
EPW = 10320
EPAD = EPW * NW
CH1 = 1032
CH2 = 240
RPS = NPAD // NS

_mesh = plsc.VectorSubcoreMesh(
    core_axis_name="c", subcore_axis_name="s", num_cores=NC, num_subcores=NS)
_sc_params = pltpu.CompilerParams(needs_layout_passes=False)


def _pre_body(x_ref, w_ref, as_ref, ad_ref, h_ref, asrc_ref, adst_ref, c_ref):
    h = jnp.dot(x_ref[...], w_ref[...], preferred_element_type=jnp.float32)
    h_ref[...] = h
    a_s = jnp.dot(h, as_ref[...], preferred_element_type=jnp.float32)
    a_d = jnp.dot(h, ad_ref[...], preferred_element_type=jnp.float32)
    asrc_ref[...] = a_s
    adst_ref[...] = a_d
    cm = (jnp.max(a_s, axis=0, keepdims=True)
          + jnp.max(a_d, axis=0, keepdims=True))
    c_ref[...] = jnp.where(cm >= 0.0, cm, 0.2 * cm)


def _stage1(xpad, w_gat, as_mat, ad_mat):
    return pl.pallas_call(
        _pre_body,
        out_shape=[
            jax.ShapeDtypeStruct((NPAD, DIM), jnp.float32),
            jax.ShapeDtypeStruct((NPAD, HEADS), jnp.float32),
            jax.ShapeDtypeStruct((NPAD, HEADS), jnp.float32),
            jax.ShapeDtypeStruct((1, HEADS), jnp.float32),
        ],
    )(xpad, w_gat, as_mat, ad_mat)


def _sc_pass1(src_ref, dst_ref, asrc_hbm, adst_hbm, cvec_hbm, zden_hbm,
              ex_hbm, den_hbm,
              tab_s, tab_d, si_v, di_v, ex_v, fidx_v, c_v, den_sh):
    cid = lax.axis_index("c")
    sid = lax.axis_index("s")
    wid = sid * NC + cid
    base = wid * EPW

    pltpu.sync_copy(asrc_hbm, tab_s)
    pltpu.sync_copy(adst_hbm, tab_d)
    pltpu.sync_copy(cvec_hbm, c_v)
    pltpu.sync_copy(zden_hbm.at[pl.ds(sid * RPS * HEADS, RPS * HEADS)],
                    den_sh.at[pl.ds(sid * RPS * HEADS, RPS * HEADS)])
    plsc.subcore_barrier()

    cvec = c_v[...]
    iota = lax.iota(jnp.int32, NLANE)
    pat4 = iota // HEADS
    lane4 = iota % HEADS

    def chunk_body(k, carry):
        off = base + k * CH1
        pltpu.sync_copy(src_ref.at[pl.ds(off, CH1)], si_v)
        pltpu.sync_copy(dst_ref.at[pl.ds(off, CH1)], di_v)

        def grp(g, c2):
            rows = pat4 + g * HEADS
            s_rep = plsc.load_gather(si_v, [rows])
            d_rep = plsc.load_gather(di_v, [rows])
            e = (plsc.load_gather(tab_s, [s_rep * HEADS + lane4])
                 + plsc.load_gather(tab_d, [d_rep * HEADS + lane4]))
            e = jnp.maximum(e, 0.0) + 0.2 * jnp.minimum(e, 0.0)
            ex = jnp.exp(e - cvec)
            sl = pl.ds(g * NLANE, NLANE)
            ex_v[sl] = ex
            fidx_v[sl] = d_rep * HEADS + lane4
            return c2

        lax.fori_loop(0, CH1 // HEADS, grp, 0)
        pltpu.sync_copy(ex_v, ex_hbm.at[pl.ds(off * HEADS, CH1 * HEADS)])
        pltpu.sync_copy(ex_v, den_sh.at[fidx_v], add=True)
        return carry

    lax.fori_loop(0, EPW // CH1, chunk_body, 0)
    plsc.subcore_barrier()
    pltpu.sync_copy(den_sh.at[pl.ds(sid * RPS * HEADS, RPS * HEADS)],
                    den_hbm.at[cid, pl.ds(sid * RPS * HEADS, RPS * HEADS)])


_sc_pass1_call = functools.partial(
    pl.kernel,
    out_type=(
        jax.ShapeDtypeStruct((EPAD * HEADS,), jnp.float32),
        jax.ShapeDtypeStruct((NC, NPAD * HEADS), jnp.float32),
    ),
    mesh=_mesh,
    compiler_params=_sc_params,
    scratch_types=[
        pltpu.VMEM((NPAD * HEADS,), jnp.float32),
        pltpu.VMEM((NPAD * HEADS,), jnp.float32),
        pltpu.VMEM((CH1,), jnp.int32),
        pltpu.VMEM((CH1,), jnp.int32),
        pltpu.VMEM((CH1 * HEADS,), jnp.float32),
        pltpu.VMEM((CH1 * HEADS,), jnp.int32),
        pltpu.VMEM((NLANE,), jnp.float32),
        pltpu.VMEM_SHARED((NPAD * HEADS,), jnp.float32),
    ],
)(_sc_pass1)


def _sc_pass2(src_ref, dst_ref, h_hbm, ex_hbm, zout_hbm,
              out_hbm,
              si_v, di_v, hv, exv, out_sh, sem1):
    cid = lax.axis_index("c")
    sid = lax.axis_index("s")
    wid = sid * NC + cid
    base = wid * EPW

    pltpu.sync_copy(zout_hbm.at[pl.ds(sid * RPS, RPS), :],
                    out_sh.at[pl.ds(sid * RPS, RPS), :])
    plsc.subcore_barrier()

    def chunk_body(k, carry):
        off = base + k * CH2
        pltpu.sync_copy(src_ref.at[pl.ds(off, CH2)], si_v)
        pltpu.sync_copy(dst_ref.at[pl.ds(off, CH2)], di_v)
        pltpu.async_copy(h_hbm.at[si_v], hv, sem1).wait()
        pltpu.sync_copy(ex_hbm.at[pl.ds(off * HEADS, CH2 * HEADS)], exv)

        def agrp(g, c2):
            for j in range(4):
                e = g * 4 + j
                for hp in range(HEADS):
                    lane = jnp.full((NLANE,), g * NLANE + j * HEADS + hp,
                                    jnp.int32)
                    a = plsc.load_gather(exv, [lane])
                    for half in range(2):
                        sl = pl.ds((hp * 2 + half) * NLANE, NLANE)
                        hv[e, sl] = hv[e, sl] * a
            return c2

        lax.fori_loop(0, CH2 // 4, agrp, 0)
        pltpu.sync_copy(hv, out_sh.at[di_v], add=True)
        return carry

    lax.fori_loop(0, EPW // CH2, chunk_body, 0)
    plsc.subcore_barrier()
    pltpu.sync_copy(out_sh.at[pl.ds(sid * RPS, RPS), :],
                    out_hbm.at[cid, pl.ds(sid * RPS, RPS), :])


_sc_pass2_call = functools.partial(
    pl.kernel,
    out_type=jax.ShapeDtypeStruct((NC, NPAD, DIM), jnp.float32),
    mesh=_mesh,
    compiler_params=_sc_params,
    scratch_types=[
        pltpu.VMEM((CH2,), jnp.int32),
        pltpu.VMEM((CH2,), jnp.int32),
        pltpu.VMEM((CH2, DIM), jnp.float32),
        pltpu.VMEM((CH2 * HEADS,), jnp.float32),
        pltpu.VMEM_SHARED((NPAD, DIM), jnp.float32),
        pltpu.SemaphoreType.DMA,
    ],
)(_sc_pass2)


def _ln(u, gamma, beta):
    mu = jnp.mean(u, axis=-1, keepdims=True)
    var = jnp.mean((u - mu) ** 2, axis=-1, keepdims=True)
    return (u - mu) * jax.lax.rsqrt(var + 1e-5) * gamma + beta


def _post_body(x_ref, p_ref, d_ref, r_ref, bg_ref, g_ref, b_ref, w1_ref,
               b1_ref, w2_ref, b2_ref, o_ref):
    x = x_ref[...]
    rden = 1.0 / (d_ref[0] + d_ref[1] + 1e-16)
    rdx = jnp.dot(rden, r_ref[...], preferred_element_type=jnp.float32)
    gat = (p_ref[0] + p_ref[1]) * rdx + bg_ref[...]
    h1 = _ln(x + gat, g_ref[...], b_ref[...])
    z = jnp.dot(h1, w1_ref[...], preferred_element_type=jnp.float32) + b1_ref[...]
    f = 0.5 * z * (1.0 + lax.erf(z * (2.0 ** -0.5)))
    ffn = jnp.dot(f, w2_ref[...], preferred_element_type=jnp.float32) + b2_ref[...]
    o_ref[...] = _ln(h1 + ffn, g_ref[...], b_ref[...]) + x


def _stage4(xpad, parts, den, rexp, b_gat, gamma, beta, w1, b1, w2, b2):
    blk = 1024
    grid = (NPAD // blk,)
    return pl.pallas_call(
        _post_body,
        grid=grid,
        in_specs=[
            pl.BlockSpec((blk, DIM), lambda i: (i, 0)),
            pl.BlockSpec((NC, blk, DIM), lambda i: (0, i, 0)),
            pl.BlockSpec((NC, blk, HEADS), lambda i: (0, i, 0)),
            pl.BlockSpec((HEADS, DIM), lambda i: (0, 0)),
            pl.BlockSpec((1, DIM), lambda i: (0, 0)),
            pl.BlockSpec((1, DIM), lambda i: (0, 0)),
            pl.BlockSpec((1, DIM), lambda i: (0, 0)),
            pl.BlockSpec((DIM, 4 * DIM), lambda i: (0, 0)),
            pl.BlockSpec((1, 4 * DIM), lambda i: (0, 0)),
            pl.BlockSpec((4 * DIM, DIM), lambda i: (0, 0)),
            pl.BlockSpec((1, DIM), lambda i: (0, 0)),
        ],
        out_specs=pl.BlockSpec((blk, DIM), lambda i: (i, 0)),
        out_shape=jax.ShapeDtypeStruct((NPAD, DIM), jnp.float32),
    )(xpad, parts, den, rexp, b_gat.reshape(1, DIM), gamma.reshape(1, DIM),
      beta.reshape(1, DIM), w1, b1.reshape(1, 4 * DIM), w2,
      b2.reshape(1, DIM))


def kernel(x, edge_index, W_gat, att_src, att_dst, b_gat, gamma, beta,
           W1, b1, W2, b2):
    xpad = jnp.pad(x, ((0, NPAD - N_NODES), (0, 0)))
    loop = jnp.arange(N_NODES, dtype=edge_index.dtype)
    padE = jnp.full((EPAD - ETOT,), PADROW, dtype=edge_index.dtype)
    src = jnp.concatenate([edge_index[0], loop, padE])
    dst = jnp.concatenate([edge_index[1], loop, padE])

    eye = jnp.eye(HEADS, dtype=jnp.float32)
    as_mat = (att_src[:, :, None] * eye[:, None, :]).reshape(DIM, HEADS)
    ad_mat = (att_dst[:, :, None] * eye[:, None, :]).reshape(DIM, HEADS)
    rexp = jnp.kron(eye, jnp.ones((1, HEAD_DIM), jnp.float32))

    h, a_s, a_d, crow = _stage1(xpad, W_gat, as_mat, ad_mat)
    cvec = jnp.tile(crow.reshape(HEADS), NLANE // HEADS)

    zden = jnp.zeros((NPAD * HEADS,), jnp.float32)
    ex, den = _sc_pass1_call(src, dst, a_s.reshape(-1), a_d.reshape(-1),
                             cvec, zden)

    zout = jnp.zeros((NPAD, DIM), jnp.float32)
    parts = _sc_pass2_call(src, dst, h, ex, zout)

    out = _stage4(xpad, parts, den.reshape(NC, NPAD, HEADS), rexp,
                  b_gat, gamma, beta, W1, b1, W2, b2)
    return out[:N_NODES]


# speedup vs baseline: 65.8877x; 1.0351x over previous
import functools

import jax
import jax.numpy as jnp
from jax import lax
from jax.experimental import pallas as pl
from jax.experimental.pallas import tpu as pltpu
from jax.experimental.pallas import tpu_sc as plsc

DIM = 128
HEADS = 4
HEAD_DIM = DIM // HEADS
N_NODES = 10000
N_EDGES = 320000

NPAD = 10240
PADROW = N_NODES
ETOT = N_EDGES + N_NODES
NC, NS, NLANE = 2, 16, 16
NW = NC * NS
EPW = 10320
EPAD = EPW * NW
CH1 = 1032
CH2 = 120
RPS = NPAD // NS

_mesh = plsc.VectorSubcoreMesh(
    core_axis_name="c", subcore_axis_name="s", num_cores=NC, num_subcores=NS)
_sc_params = pltpu.CompilerParams(needs_layout_passes=False)


def _pre_body(x_ref, w_ref, as_ref, ad_ref, h_ref, asrc_ref, adst_ref, c_ref):
    h = jnp.dot(x_ref[...], w_ref[...], preferred_element_type=jnp.float32)
    h_ref[...] = h
    a_s = jnp.dot(h, as_ref[...], preferred_element_type=jnp.float32)
    a_d = jnp.dot(h, ad_ref[...], preferred_element_type=jnp.float32)
    asrc_ref[...] = a_s
    adst_ref[...] = a_d
    cm = (jnp.max(a_s, axis=0, keepdims=True)
          + jnp.max(a_d, axis=0, keepdims=True))
    c_ref[...] = jnp.where(cm >= 0.0, cm, 0.2 * cm)


def _stage1(xpad, w_gat, as_mat, ad_mat):
    return pl.pallas_call(
        _pre_body,
        out_shape=[
            jax.ShapeDtypeStruct((NPAD, DIM), jnp.float32),
            jax.ShapeDtypeStruct((NPAD, HEADS), jnp.float32),
            jax.ShapeDtypeStruct((NPAD, HEADS), jnp.float32),
            jax.ShapeDtypeStruct((1, HEADS), jnp.float32),
        ],
    )(xpad, w_gat, as_mat, ad_mat)


def _sc_pass1(src_ref, dst_ref, asrc_hbm, adst_hbm, cvec_hbm, zden_hbm,
              ex_hbm, den_hbm,
              tab_s, tab_d, si_v, di_v, ex_v, fidx_v, c_v, den_sh):
    cid = lax.axis_index("c")
    sid = lax.axis_index("s")
    wid = sid * NC + cid
    base = wid * EPW

    pltpu.sync_copy(asrc_hbm, tab_s)
    pltpu.sync_copy(adst_hbm, tab_d)
    pltpu.sync_copy(cvec_hbm, c_v)
    pltpu.sync_copy(zden_hbm.at[pl.ds(sid * RPS * HEADS, RPS * HEADS)],
                    den_sh.at[pl.ds(sid * RPS * HEADS, RPS * HEADS)])
    plsc.subcore_barrier()

    cvec = c_v[...]
    iota = lax.iota(jnp.int32, NLANE)
    pat4 = iota // HEADS
    lane4 = iota % HEADS

    def chunk_body(k, carry):
        off = base + k * CH1
        pltpu.sync_copy(src_ref.at[pl.ds(off, CH1)], si_v)
        pltpu.sync_copy(dst_ref.at[pl.ds(off, CH1)], di_v)

        def grp(g, c2):
            rows = pat4 + g * HEADS
            s_rep = plsc.load_gather(si_v, [rows])
            d_rep = plsc.load_gather(di_v, [rows])
            e = (plsc.load_gather(tab_s, [s_rep * HEADS + lane4])
                 + plsc.load_gather(tab_d, [d_rep * HEADS + lane4]))
            e = jnp.maximum(e, 0.0) + 0.2 * jnp.minimum(e, 0.0)
            ex = jnp.exp(e - cvec)
            sl = pl.ds(g * NLANE, NLANE)
            ex_v[sl] = ex
            fidx_v[sl] = d_rep * HEADS + lane4
            return c2

        lax.fori_loop(0, CH1 // HEADS, grp, 0)
        pltpu.sync_copy(ex_v, ex_hbm.at[pl.ds(off * HEADS, CH1 * HEADS)])
        pltpu.sync_copy(ex_v, den_sh.at[fidx_v], add=True)
        return carry

    lax.fori_loop(0, EPW // CH1, chunk_body, 0)
    plsc.subcore_barrier()
    pltpu.sync_copy(den_sh.at[pl.ds(sid * RPS * HEADS, RPS * HEADS)],
                    den_hbm.at[cid, pl.ds(sid * RPS * HEADS, RPS * HEADS)])


_sc_pass1_call = functools.partial(
    pl.kernel,
    out_type=(
        jax.ShapeDtypeStruct((EPAD * HEADS,), jnp.float32),
        jax.ShapeDtypeStruct((NC, NPAD * HEADS), jnp.float32),
    ),
    mesh=_mesh,
    compiler_params=_sc_params,
    scratch_types=[
        pltpu.VMEM((NPAD * HEADS,), jnp.float32),
        pltpu.VMEM((NPAD * HEADS,), jnp.float32),
        pltpu.VMEM((CH1,), jnp.int32),
        pltpu.VMEM((CH1,), jnp.int32),
        pltpu.VMEM((CH1 * HEADS,), jnp.float32),
        pltpu.VMEM((CH1 * HEADS,), jnp.int32),
        pltpu.VMEM((NLANE,), jnp.float32),
        pltpu.VMEM_SHARED((NPAD * HEADS,), jnp.float32),
    ],
)(_sc_pass1)


def _sc_pass2(src_ref, dst_ref, h_hbm, ex_hbm, zout_hbm,
              out_hbm,
              si0, di0, hv0, ex0, si1, di1, hv1, ex1, out_sh, sem0, sem1):
    cid = lax.axis_index("c")
    sid = lax.axis_index("s")
    wid = sid * NC + cid
    base = wid * EPW
    nch = EPW // CH2

    pltpu.sync_copy(zout_hbm.at[pl.ds(sid * RPS, RPS), :],
                    out_sh.at[pl.ds(sid * RPS, RPS), :])
    plsc.subcore_barrier()

    def stage(k, si_v, di_v, exv, hv, sem):
        off = base + k * CH2
        pltpu.sync_copy(src_ref.at[pl.ds(off, CH2)], si_v)
        pltpu.sync_copy(dst_ref.at[pl.ds(off, CH2)], di_v)
        pltpu.sync_copy(ex_hbm.at[pl.ds(off * HEADS, CH2 * HEADS)], exv)
        pltpu.async_copy(h_hbm.at[si_v], hv, sem)

    def work(si_v, di_v, exv, hv, sem):
        pltpu.make_async_copy(h_hbm.at[si_v], hv, sem).wait()

        def agrp(g, c2):
            for j in range(4):
                e = g * 4 + j
                for hp in range(HEADS):
                    lane = jnp.full((NLANE,), g * NLANE + j * HEADS + hp,
                                    jnp.int32)
                    a = plsc.load_gather(exv, [lane])
                    for half in range(2):
                        sl = pl.ds((hp * 2 + half) * NLANE, NLANE)
                        hv[e, sl] = hv[e, sl] * a
            return c2

        lax.fori_loop(0, CH2 // 4, agrp, 0)
        pltpu.sync_copy(hv, out_sh.at[di_v], add=True)

    stage(0, si0, di0, ex0, hv0, sem0)

    def pair_body(kk, carry):
        k0 = 2 * kk
        stage(k0 + 1, si1, di1, ex1, hv1, sem1)
        work(si0, di0, ex0, hv0, sem0)

        @pl.when(k0 + 2 < nch)
        def _():
            stage(k0 + 2, si0, di0, ex0, hv0, sem0)

        work(si1, di1, ex1, hv1, sem1)
        return carry

    lax.fori_loop(0, nch // 2, pair_body, 0)
    plsc.subcore_barrier()
    pltpu.sync_copy(out_sh.at[pl.ds(sid * RPS, RPS), :],
                    out_hbm.at[cid, pl.ds(sid * RPS, RPS), :])


_sc_pass2_call = functools.partial(
    pl.kernel,
    out_type=jax.ShapeDtypeStruct((NC, NPAD, DIM), jnp.float32),
    mesh=_mesh,
    compiler_params=_sc_params,
    scratch_types=[
        pltpu.VMEM((CH2,), jnp.int32),
        pltpu.VMEM((CH2,), jnp.int32),
        pltpu.VMEM((CH2, DIM), jnp.float32),
        pltpu.VMEM((CH2 * HEADS,), jnp.float32),
        pltpu.VMEM((CH2,), jnp.int32),
        pltpu.VMEM((CH2,), jnp.int32),
        pltpu.VMEM((CH2, DIM), jnp.float32),
        pltpu.VMEM((CH2 * HEADS,), jnp.float32),
        pltpu.VMEM_SHARED((NPAD, DIM), jnp.float32),
        pltpu.SemaphoreType.DMA,
        pltpu.SemaphoreType.DMA,
    ],
)(_sc_pass2)


def _ln(u, gamma, beta):
    mu = jnp.mean(u, axis=-1, keepdims=True)
    var = jnp.mean((u - mu) ** 2, axis=-1, keepdims=True)
    return (u - mu) * jax.lax.rsqrt(var + 1e-5) * gamma + beta


def _post_body(x_ref, p_ref, d_ref, r_ref, bg_ref, g_ref, b_ref, w1_ref,
               b1_ref, w2_ref, b2_ref, o_ref):
    x = x_ref[...]
    rden = 1.0 / (d_ref[0] + d_ref[1] + 1e-16)
    rdx = jnp.dot(rden, r_ref[...], preferred_element_type=jnp.float32)
    gat = (p_ref[0] + p_ref[1]) * rdx + bg_ref[...]
    h1 = _ln(x + gat, g_ref[...], b_ref[...])
    z = jnp.dot(h1, w1_ref[...], preferred_element_type=jnp.float32) + b1_ref[...]
    f = 0.5 * z * (1.0 + lax.erf(z * (2.0 ** -0.5)))
    ffn = jnp.dot(f, w2_ref[...], preferred_element_type=jnp.float32) + b2_ref[...]
    o_ref[...] = _ln(h1 + ffn, g_ref[...], b_ref[...]) + x


def _stage4(xpad, parts, den, rexp, b_gat, gamma, beta, w1, b1, w2, b2):
    blk = 1024
    grid = (NPAD // blk,)
    return pl.pallas_call(
        _post_body,
        grid=grid,
        in_specs=[
            pl.BlockSpec((blk, DIM), lambda i: (i, 0)),
            pl.BlockSpec((NC, blk, DIM), lambda i: (0, i, 0)),
            pl.BlockSpec((NC, blk, HEADS), lambda i: (0, i, 0)),
            pl.BlockSpec((HEADS, DIM), lambda i: (0, 0)),
            pl.BlockSpec((1, DIM), lambda i: (0, 0)),
            pl.BlockSpec((1, DIM), lambda i: (0, 0)),
            pl.BlockSpec((1, DIM), lambda i: (0, 0)),
            pl.BlockSpec((DIM, 4 * DIM), lambda i: (0, 0)),
            pl.BlockSpec((1, 4 * DIM), lambda i: (0, 0)),
            pl.BlockSpec((4 * DIM, DIM), lambda i: (0, 0)),
            pl.BlockSpec((1, DIM), lambda i: (0, 0)),
        ],
        out_specs=pl.BlockSpec((blk, DIM), lambda i: (i, 0)),
        out_shape=jax.ShapeDtypeStruct((NPAD, DIM), jnp.float32),
    )(xpad, parts, den, rexp, b_gat.reshape(1, DIM), gamma.reshape(1, DIM),
      beta.reshape(1, DIM), w1, b1.reshape(1, 4 * DIM), w2,
      b2.reshape(1, DIM))


def kernel(x, edge_index, W_gat, att_src, att_dst, b_gat, gamma, beta,
           W1, b1, W2, b2):
    xpad = jnp.pad(x, ((0, NPAD - N_NODES), (0, 0)))
    loop = jnp.arange(N_NODES, dtype=edge_index.dtype)
    padE = jnp.full((EPAD - ETOT,), PADROW, dtype=edge_index.dtype)
    src = jnp.concatenate([edge_index[0], loop, padE])
    dst = jnp.concatenate([edge_index[1], loop, padE])

    eye = jnp.eye(HEADS, dtype=jnp.float32)
    as_mat = (att_src[:, :, None] * eye[:, None, :]).reshape(DIM, HEADS)
    ad_mat = (att_dst[:, :, None] * eye[:, None, :]).reshape(DIM, HEADS)
    rexp = jnp.kron(eye, jnp.ones((1, HEAD_DIM), jnp.float32))

    h, a_s, a_d, crow = _stage1(xpad, W_gat, as_mat, ad_mat)
    cvec = jnp.tile(crow.reshape(HEADS), NLANE // HEADS)

    zden = jnp.zeros((NPAD * HEADS,), jnp.float32)
    ex, den = _sc_pass1_call(src, dst, a_s.reshape(-1), a_d.reshape(-1),
                             cvec, zden)

    zout = jnp.zeros((NPAD, DIM), jnp.float32)
    parts = _sc_pass2_call(src, dst, h, ex, zout)

    out = _stage4(xpad, parts, den.reshape(NC, NPAD, HEADS), rexp,
                  b_gat, gamma, beta, W1, b1, W2, b2)
    return out[:N_NODES]
